# baseline (speedup 1.0000x reference)
import functools

import jax
import jax.numpy as jnp
from jax import lax
from jax.experimental import pallas as pl
from jax.experimental.pallas import tpu as pltpu
from jax.experimental.pallas import tpu_sc as plsc

B = 4096
F = 26
V = 100000
D = 16

NC = 2
NS = 16
L = 16
NW = NC * NS
BPW = B // NW
NG = BPW // L
NK = D * F
LAG = 32


@functools.cache
def _build_fm_sc():
    mesh = plsc.VectorSubcoreMesh(
        core_axis_name="c", subcore_axis_name="s", num_cores=NC, num_subcores=NS
    )

    @functools.partial(
        pl.kernel,
        out_type=jax.ShapeDtypeStruct((B,), jnp.float32),
        mesh=mesh,
        scratch_types=[
            pltpu.VMEM((NK, BPW), jnp.int32),
            pltpu.VMEM((NK * BPW,), jnp.float32),
            pltpu.VMEM((F, BPW), jnp.float32),
            pltpu.VMEM((BPW,), jnp.float32),
            pltpu.SemaphoreType.DMA,
            pltpu.SemaphoreType.DMA,
        ],
        compiler_params=pltpu.CompilerParams(
            needs_layout_passes=False, use_tc_tiling_on_sc=False
        ),
    )
    def _fm_sc(idx_hbm, emb1_hbm, emb2_hbm, out_hbm,
               idx_v, rows_v, first_v, out_v, sem, sem1):
        w = lax.axis_index("c") * NS + lax.axis_index("s")

        pltpu.sync_copy(idx_hbm.at[w], idx_v)

        def _issue1(f, carry):
            pltpu.make_async_copy(emb1_hbm.at[idx_v.at[f]], first_v.at[f], sem1).start()
            return carry

        lax.fori_loop(0, F, _issue1, 0)

        def _fire(k):
            dst = rows_v.at[pl.ds(k * BPW, BPW)]
            pltpu.make_async_copy(emb2_hbm.at[idx_v.at[k]], dst, sem).start()

        def _drain(k):
            dst = rows_v.at[pl.ds(k * BPW, BPW)]
            pltpu.make_async_copy(emb2_hbm.at[idx_v.at[k]], dst, sem).wait()

        def _prologue(k, carry):
            _fire(k)
            return carry

        lax.fori_loop(0, LAG, _prologue, 0)

        def _steady(k, carry):
            _fire(k)
            _drain(k - LAG)
            return carry

        lax.fori_loop(LAG, NK, _steady, 0)

        def _epilogue(k, carry):
            _drain(k)
            return carry

        lax.fori_loop(NK - LAG, NK, _epilogue, 0)

        def _drain1(f, carry):
            pltpu.make_async_copy(emb1_hbm.at[idx_v.at[f]], first_v.at[f], sem1).wait()
            return carry

        lax.fori_loop(0, F, _drain1, 0)

        def _dstep(d, carry):
            for g in range(NG):
                base = d * F * BPW + g * L
                acc = rows_v[pl.ds(base, L)]
                acc2 = acc * acc
                for f in range(1, F):
                    v = rows_v[pl.ds(base + f * BPW, L)]
                    acc = acc + v
                    acc2 = acc2 + v * v
                carry[g] = carry[g] + (acc * acc - acc2) * 0.5
            return carry

        p1 = []
        for g in range(NG):
            t = first_v[0, pl.ds(g * L, L)]
            for f in range(1, F):
                t = t + first_v[f, pl.ds(g * L, L)]
            p1.append(t)

        res = lax.fori_loop(0, D, _dstep, p1)
        for g in range(NG):
            out_v[pl.ds(g * L, L)] = res[g]

        pltpu.sync_copy(out_v, out_hbm.at[pl.ds(w * BPW, BPW)])

    return _fm_sc


def kernel(sparse_idx, emb_first, emb_second):
    flat_idx = sparse_idx + (jnp.arange(F, dtype=sparse_idx.dtype) * V)[None, :]
    per_w = flat_idx.reshape(NW, BPW, F).transpose(0, 2, 1)
    d_off = (jnp.arange(D, dtype=jnp.int32) * (F * V))[None, :, None, None]
    cube = (per_w[:, None, :, :] + d_off).reshape(NW, NK, BPW)
    emb2_lin = emb_second.T.reshape(-1)
    out = _build_fm_sc()(cube, emb_first.reshape(-1), emb2_lin)
    return out[:, None]


# speedup vs baseline: 2.9968x; 2.9968x over previous
import functools

import jax
import jax.numpy as jnp
from jax import lax
from jax.experimental import pallas as pl
from jax.experimental.pallas import tpu as pltpu
from jax.experimental.pallas import tpu_sc as plsc

B = 4096
F = 26
V = 100000
D = 16

NC = 2
NS = 16
L = 16
NW = NC * NS
BPW = B // NW
NG = BPW // L


@functools.cache
def _build_fm_sc():
    mesh = plsc.VectorSubcoreMesh(
        core_axis_name="c", subcore_axis_name="s", num_cores=NC, num_subcores=NS
    )

    @functools.partial(
        pl.kernel,
        out_type=jax.ShapeDtypeStruct((B,), jnp.float32),
        mesh=mesh,
        scratch_types=[
            pltpu.VMEM((F, BPW), jnp.int32),
            pltpu.VMEM((F, BPW, D), jnp.float32),
            pltpu.VMEM((F, BPW), jnp.float32),
            pltpu.VMEM((BPW * D,), jnp.float32),
            pltpu.VMEM((BPW,), jnp.float32),
            pltpu.SemaphoreType.DMA,
            pltpu.SemaphoreType.DMA,
        ],
        compiler_params=pltpu.CompilerParams(
            needs_layout_passes=False, use_tc_tiling_on_sc=False
        ),
    )
    def _fm_sc(idx_hbm, emb1_hbm, emb2_hbm, out_hbm,
               idx_v, rows_v, first_v, p2_v, out_v, sem, sem1):
        w = lax.axis_index("c") * NS + lax.axis_index("s")

        pltpu.sync_copy(idx_hbm.at[w], idx_v)

        for f in range(F):
            pltpu.make_async_copy(
                emb2_hbm.at[idx_v.at[f]], rows_v.at[f], sem
            ).start()
        for f in range(F):
            pltpu.make_async_copy(
                emb1_hbm.at[idx_v.at[f]], first_v.at[f], sem1
            ).start()
        for f in range(F):
            pltpu.make_async_copy(
                emb2_hbm.at[idx_v.at[f]], rows_v.at[f], sem
            ).wait()
        for f in range(F):
            pltpu.make_async_copy(
                emb1_hbm.at[idx_v.at[f]], first_v.at[f], sem1
            ).wait()

        def _sample(s, carry):
            acc = rows_v[0, s]
            acc2 = acc * acc
            for f in range(1, F):
                v = rows_v[f, s]
                acc = acc + v
                acc2 = acc2 + v * v
            p2_v[pl.ds(s * D, D)] = (acc * acc - acc2) * 0.5
            return carry

        lax.fori_loop(0, BPW, _sample, 0)

        lanes = lax.iota(jnp.int32, L)
        for g in range(NG):
            t = first_v[0, pl.ds(g * L, L)]
            for f in range(1, F):
                t = t + first_v[f, pl.ds(g * L, L)]
            sample_base = (lanes + g * L) * D
            for d in range(D):
                t = t + plsc.load_gather(p2_v, [sample_base + d])
            out_v[pl.ds(g * L, L)] = t

        pltpu.sync_copy(out_v, out_hbm.at[pl.ds(w * BPW, BPW)])

    return _fm_sc


def kernel(sparse_idx, emb_first, emb_second):
    flat_idx = sparse_idx + (jnp.arange(F, dtype=sparse_idx.dtype) * V)[None, :]
    per_w = flat_idx.reshape(NW, BPW, F).transpose(0, 2, 1)
    out = _build_fm_sc()(per_w, emb_first.reshape(-1), emb_second)
    return out[:, None]
